# baseline (speedup 1.0000x reference)
import functools

import jax
import jax.numpy as jnp
import numpy as np
from jax import lax
from jax.experimental import pallas as pl
from jax.experimental.pallas import tpu as pltpu
from jax.experimental.pallas import tpu_sc as plsc

_PROB = 0.5
_NUM_PREFIX = 1
_B, _T, _D = 128, 577, 768
_NP = _T - _NUM_PREFIX
_NK = max(1, int(_NP * (1.0 - _PROB)))
_ROWS = _NUM_PREFIX + _NK
_NW = 32
_BPW = _B // _NW
_CH = 72
_NCH = 4
_TAIL = _ROWS - _NCH * _CH


@functools.lru_cache(maxsize=1)
def _row_indices() -> np.ndarray:
    with jax.ensure_compile_time_eval():
        rand = jax.random.normal(jax.random.key(42), (_B, _NP), dtype=jnp.float32)
        order = jnp.argsort(rand, axis=-1)
        keep = jnp.sort(order[:, :_NK], axis=-1) + _NUM_PREFIX
        full = jnp.concatenate(
            [jnp.zeros((_B, _NUM_PREFIX), keep.dtype), keep], axis=1)
    return np.asarray(full).astype(np.int32)


def _sc_gather(x, src_main, src_tail, dst_tail):
    mesh = plsc.VectorSubcoreMesh(core_axis_name="c", subcore_axis_name="s")

    @functools.partial(
        pl.kernel,
        mesh=mesh,
        out_type=jax.ShapeDtypeStruct((_B, _ROWS, _D), jnp.float32),
        compiler_params=pltpu.CompilerParams(use_tc_tiling_on_sc=False),
        scratch_types=[
            pltpu.VMEM((_BPW, _NCH, _CH), jnp.int32),
            pltpu.VMEM((_BPW, 1, _TAIL), jnp.int32),
            pltpu.VMEM((1, _TAIL), jnp.int32),
            pltpu.VMEM((_CH, _D), jnp.float32),
            pltpu.VMEM((_CH, _D), jnp.float32),
            pltpu.VMEM((_TAIL, _D), jnp.float32),
            pltpu.SemaphoreType.DMA,
            pltpu.SemaphoreType.DMA,
            pltpu.SemaphoreType.DMA,
            pltpu.SemaphoreType.DMA,
            pltpu.SemaphoreType.DMA,
            pltpu.SemaphoreType.DMA,
        ],
    )
    def gather_rows(x_hbm, srcm_hbm, srct_hbm, dstt_hbm, out_hbm,
                    idxm_v, idxt_v, widx_v, buf0, buf1, tbuf,
                    gsem0, gsem1, gsemt, wsem0, wsem1, wsemt):
        wid = lax.axis_index("s") * 2 + lax.axis_index("c")
        base = wid * _BPW
        pltpu.sync_copy(srcm_hbm.at[pl.ds(base, _BPW)], idxm_v)
        pltpu.sync_copy(srct_hbm.at[pl.ds(base, _BPW)], idxt_v)
        pltpu.sync_copy(dstt_hbm, widx_v)

        bufs = {0: buf0, 1: buf1, "t": tbuf}
        gsems = {0: gsem0, 1: gsem1, "t": gsemt}
        wsems = {0: wsem0, 1: wsem1, "t": wsemt}

        jobs = []
        for i in range(_BPW):
            for g in range(_NCH):
                jobs.append(("m", i, g, (i * _NCH + g) % 2))
            jobs.append(("t", i, 0, "t"))

        gds = [None] * len(jobs)
        last_write = {0: None, 1: None, "t": None}

        def idx_ref(job):
            kind, i, g, _ = job
            if kind == "m":
                return idxm_v.at[i].at[g]
            return idxt_v.at[i].at[0]

        def start_gather(j):
            kind, i, g, tag = jobs[j]
            if last_write[tag] is not None:
                last_write[tag].wait()
                last_write[tag] = None
            gds[j] = pltpu.async_copy(
                x_hbm.at[base + i].at[idx_ref(jobs[j])], bufs[tag], gsems[tag])

        start_gather(0)
        for j in range(len(jobs)):
            if j + 1 < len(jobs):
                start_gather(j + 1)
            gds[j].wait()
            kind, i, g, tag = jobs[j]
            if kind == "m":
                last_write[tag] = pltpu.async_copy(
                    bufs[tag], out_hbm.at[base + i, pl.ds(g * _CH, _CH)],
                    wsems[tag])
            else:
                last_write[tag] = pltpu.async_copy(
                    tbuf, out_hbm.at[base + i].at[widx_v.at[0]], wsems[tag])
        for tag in (0, 1, "t"):
            if last_write[tag] is not None:
                last_write[tag].wait()

    return gather_rows(x, src_main, src_tail, dst_tail)


def kernel(inputs):
    x = inputs
    rows = _row_indices()
    src_main = jnp.asarray(rows[:, : _NCH * _CH].reshape(_B, _NCH, _CH))
    src_tail = jnp.asarray(rows[:, _NCH * _CH :].reshape(_B, 1, _TAIL))
    dst_tail = jnp.asarray(
        np.arange(_NCH * _CH, _ROWS, dtype=np.int32).reshape(1, _TAIL))
    return _sc_gather(x, src_main, src_tail, dst_tail)


# speedup vs baseline: 5.7039x; 5.7039x over previous
import functools

import jax
import jax.numpy as jnp
import numpy as np
from jax import lax
from jax.experimental import pallas as pl
from jax.experimental.pallas import tpu as pltpu
from jax.experimental.pallas import tpu_sc as plsc

_PROB = 0.5
_NUM_PREFIX = 1
_B, _T, _D = 128, 577, 768
_NP = _T - _NUM_PREFIX
_NK = max(1, int(_NP * (1.0 - _PROB)))
_ROWS = _NUM_PREFIX + _NK
_NW = 32
_TPW = _ROWS // _NW
_HB = _B // 2


@functools.lru_cache(maxsize=1)
def _gather_indices() -> np.ndarray:
    with jax.ensure_compile_time_eval():
        rand = jax.random.normal(jax.random.key(42), (_B, _NP), dtype=jnp.float32)
        order = jnp.argsort(rand, axis=-1)
        keep = jnp.sort(order[:, :_NK], axis=-1) + _NUM_PREFIX
        full = jnp.concatenate(
            [jnp.zeros((_B, _NUM_PREFIX), keep.dtype), keep], axis=1)
    rows = np.asarray(full).astype(np.int32)
    flat = rows.T * _B + np.arange(_B, dtype=np.int32)[None, :]
    return flat.reshape(_ROWS, 1, _B)


def _sc_gather(x2, idx3):
    mesh = plsc.VectorSubcoreMesh(core_axis_name="c", subcore_axis_name="s")

    @functools.partial(
        pl.kernel,
        mesh=mesh,
        out_type=jax.ShapeDtypeStruct((_ROWS * _B, _D), jnp.float32),
        scratch_types=[
            pltpu.VMEM((_TPW + 1, 1, _B), jnp.int32),
            pltpu.VMEM((_HB, _D), jnp.float32),
            pltpu.VMEM((_HB, _D), jnp.float32),
            pltpu.SemaphoreType.DMA,
            pltpu.SemaphoreType.DMA,
            pltpu.SemaphoreType.DMA,
            pltpu.SemaphoreType.DMA,
        ],
    )
    def gather_rows(x_hbm, idx_hbm, out_hbm, idx_v, buf0, buf1,
                    gsem0, gsem1, wsem0, wsem1):
        wid = lax.axis_index("s") * 2 + lax.axis_index("c")

        for k in range(_TPW):
            pltpu.sync_copy(idx_hbm.at[wid + _NW * k], idx_v.at[k])
        @pl.when(wid == 0)
        def _():
            pltpu.sync_copy(idx_hbm.at[_ROWS - 1], idx_v.at[_TPW])

        bufs = (buf0, buf1)
        gsems = (gsem0, gsem1)
        wsems = (wsem0, wsem1)

        jobs = [(k, h) for k in range(_TPW) for h in range(2)]
        gds = [None] * len(jobs)
        last_write = [None, None]

        def row_off(k, h):
            return pl.multiple_of((wid + _NW * k) * _B + _HB * h, _HB)

        def start_gather(j):
            k, h = jobs[j]
            tag = j % 2
            if last_write[tag] is not None:
                last_write[tag].wait()
                last_write[tag] = None
            gds[j] = pltpu.async_copy(
                x_hbm.at[idx_v.at[k].at[0].at[pl.ds(_HB * h, _HB)]],
                bufs[tag], gsems[tag])

        start_gather(0)
        for j in range(len(jobs)):
            if j + 1 < len(jobs):
                start_gather(j + 1)
            gds[j].wait()
            k, h = jobs[j]
            tag = j % 2
            last_write[tag] = pltpu.async_copy(
                bufs[tag], out_hbm.at[pl.ds(row_off(k, h), _HB)], wsems[tag])
        for tag in (0, 1):
            if last_write[tag] is not None:
                last_write[tag].wait()
                last_write[tag] = None

        @pl.when(wid == 0)
        def _():
            for h in range(2):
                pltpu.async_copy(
                    x_hbm.at[idx_v.at[_TPW].at[0].at[pl.ds(_HB * h, _HB)]],
                    bufs[h], gsems[h]).wait()
                pltpu.sync_copy(
                    bufs[h],
                    out_hbm.at[pl.ds((_ROWS - 1) * _B + _HB * h, _HB)])

    return gather_rows(x2, idx3)


def kernel(inputs):
    x = inputs
    x2 = jnp.transpose(x, (1, 0, 2)).reshape(_T * _B, _D)
    idx3 = jnp.asarray(_gather_indices())
    out2 = _sc_gather(x2, idx3)
    return jnp.transpose(out2.reshape(_ROWS, _B, _D), (1, 0, 2))


# speedup vs baseline: 5.8187x; 1.0201x over previous
import functools

import jax
import jax.numpy as jnp
import numpy as np
from jax import lax
from jax.experimental import pallas as pl
from jax.experimental.pallas import tpu as pltpu
from jax.experimental.pallas import tpu_sc as plsc

_PROB = 0.5
_NUM_PREFIX = 1
_B, _T, _D = 128, 577, 768
_NP = _T - _NUM_PREFIX
_NK = max(1, int(_NP * (1.0 - _PROB)))
_ROWS = _NUM_PREFIX + _NK
_NW = 32
_TPW = _ROWS // _NW
_NBUF = 4
_CHK = _B // _NBUF
_HB = _B // 2


@functools.lru_cache(maxsize=1)
def _gather_indices() -> np.ndarray:
    with jax.ensure_compile_time_eval():
        rand = jax.random.normal(jax.random.key(42), (_B, _NP), dtype=jnp.float32)
        order = jnp.argsort(rand, axis=-1)
        keep = jnp.sort(order[:, :_NK], axis=-1) + _NUM_PREFIX
        full = jnp.concatenate(
            [jnp.zeros((_B, _NUM_PREFIX), keep.dtype), keep], axis=1)
    rows = np.asarray(full).astype(np.int32)
    flat = rows.T * _B + np.arange(_B, dtype=np.int32)[None, :]
    return flat.reshape(_ROWS, 1, _B)


def _sc_gather(x2, idx3):
    mesh = plsc.VectorSubcoreMesh(core_axis_name="c", subcore_axis_name="s")

    @functools.partial(
        pl.kernel,
        mesh=mesh,
        out_type=jax.ShapeDtypeStruct((_ROWS * _B, _D), jnp.float32),
        scratch_types=[
            pltpu.VMEM((_TPW + 1, 1, _B), jnp.int32),
            *[pltpu.VMEM((_CHK, _D), jnp.float32) for _ in range(_NBUF)],
            *[pltpu.SemaphoreType.DMA for _ in range(2 * _NBUF)],
        ],
    )
    def gather_rows(x_hbm, idx_hbm, out_hbm, idx_v, *rest):
        bufs = rest[:_NBUF]
        gsems = rest[_NBUF:2 * _NBUF]
        wsems = rest[2 * _NBUF:]
        wid = lax.axis_index("s") * 2 + lax.axis_index("c")

        for k in range(_TPW):
            pltpu.sync_copy(idx_hbm.at[wid + _NW * k], idx_v.at[k])
        @pl.when(wid < 2)
        def _():
            pltpu.sync_copy(idx_hbm.at[_ROWS - 1], idx_v.at[_TPW])

        jobs = [(k, q) for k in range(_TPW) for q in range(_NBUF)]
        gds = [None] * len(jobs)
        last_write = [None] * _NBUF

        def row_off(k, q):
            return pl.multiple_of((wid + _NW * k) * _B + _CHK * q, _CHK)

        def start_gather(j):
            k, q = jobs[j]
            tag = j % _NBUF
            if last_write[tag] is not None:
                last_write[tag].wait()
                last_write[tag] = None
            gds[j] = pltpu.async_copy(
                x_hbm.at[idx_v.at[k].at[0].at[pl.ds(_CHK * q, _CHK)]],
                bufs[tag], gsems[tag])

        nj = len(jobs)
        for j in range(_NBUF - 1):
            start_gather(j)
        for j in range(nj):
            if j + _NBUF - 1 < nj:
                start_gather(j + _NBUF - 1)
            gds[j].wait()
            k, q = jobs[j]
            tag = j % _NBUF
            last_write[tag] = pltpu.async_copy(
                bufs[tag], out_hbm.at[pl.ds(row_off(k, q), _CHK)], wsems[tag])
        for tag in range(_NBUF):
            if last_write[tag] is not None:
                last_write[tag].wait()
                last_write[tag] = None

        for w, h in ((0, 0), (1, 1)):
            @pl.when(wid == w)
            def _(h=h):
                for p in range(2):
                    q = 2 * h + p
                    pltpu.async_copy(
                        x_hbm.at[idx_v.at[_TPW].at[0].at[pl.ds(_CHK * q, _CHK)]],
                        bufs[p], gsems[p]).wait()
                    pltpu.sync_copy(
                        bufs[p],
                        out_hbm.at[pl.ds((_ROWS - 1) * _B + _CHK * q, _CHK)])

    return gather_rows(x2, idx3)


def kernel(inputs):
    x = inputs
    x2 = jnp.transpose(x, (1, 0, 2)).reshape(_T * _B, _D)
    idx3 = jnp.asarray(_gather_indices())
    out2 = _sc_gather(x2, idx3)
    return jnp.transpose(out2.reshape(_ROWS, _B, _D), (1, 0, 2))


# speedup vs baseline: 5.8902x; 1.0123x over previous
import functools

import jax
import jax.numpy as jnp
import numpy as np
from jax import lax
from jax.experimental import pallas as pl
from jax.experimental.pallas import tpu as pltpu
from jax.experimental.pallas import tpu_sc as plsc

_PROB = 0.5
_NUM_PREFIX = 1
_B, _T, _D = 128, 577, 768
_NP = _T - _NUM_PREFIX
_NK = max(1, int(_NP * (1.0 - _PROB)))
_ROWS = _NUM_PREFIX + _NK
_NW = 32
_TPW = _ROWS // _NW
_NBUF = 2
_CHK = _B // _NBUF
_HB = _B // 2


@functools.lru_cache(maxsize=1)
def _gather_indices() -> np.ndarray:
    with jax.ensure_compile_time_eval():
        rand = jax.random.normal(jax.random.key(42), (_B, _NP), dtype=jnp.float32)
        order = jnp.argsort(rand, axis=-1)
        keep = jnp.sort(order[:, :_NK], axis=-1) + _NUM_PREFIX
        full = jnp.concatenate(
            [jnp.zeros((_B, _NUM_PREFIX), keep.dtype), keep], axis=1)
    rows = np.asarray(full).astype(np.int32)
    flat = rows.T * _B + np.arange(_B, dtype=np.int32)[None, :]
    return flat.reshape(_ROWS, 1, _B)


def _sc_gather(x2, idx3):
    mesh = plsc.VectorSubcoreMesh(core_axis_name="c", subcore_axis_name="s")

    @functools.partial(
        pl.kernel,
        mesh=mesh,
        out_type=jax.ShapeDtypeStruct((_ROWS * _B, _D), jnp.float32),
        scratch_types=[
            pltpu.VMEM((_TPW + 1, 1, _B), jnp.int32),
            *[pltpu.VMEM((_CHK, _D), jnp.float32) for _ in range(_NBUF)],
            *[pltpu.SemaphoreType.DMA for _ in range(2 * _NBUF)],
        ],
    )
    def gather_rows(x_hbm, idx_hbm, out_hbm, idx_v, *rest):
        bufs = rest[:_NBUF]
        gsems = rest[_NBUF:2 * _NBUF]
        wsems = rest[2 * _NBUF:]
        wid = lax.axis_index("s") * 2 + lax.axis_index("c")

        for k in range(_TPW):
            pltpu.sync_copy(idx_hbm.at[wid + _NW * k], idx_v.at[k])
        @pl.when(wid < 2)
        def _():
            pltpu.sync_copy(idx_hbm.at[_ROWS - 1], idx_v.at[_TPW])

        jobs = [(k, q) for k in range(_TPW) for q in range(_NBUF)]
        gds = [None] * len(jobs)
        last_write = [None] * _NBUF

        def row_off(k, q):
            return pl.multiple_of((wid + _NW * k) * _B + _CHK * q, _CHK)

        def start_gather(j):
            k, q = jobs[j]
            tag = j % _NBUF
            if last_write[tag] is not None:
                last_write[tag].wait()
                last_write[tag] = None
            gds[j] = pltpu.async_copy(
                x_hbm.at[idx_v.at[k].at[0].at[pl.ds(_CHK * q, _CHK)]],
                bufs[tag], gsems[tag])

        nj = len(jobs)
        for j in range(_NBUF - 1):
            start_gather(j)
        for j in range(nj):
            if j + _NBUF - 1 < nj:
                start_gather(j + _NBUF - 1)
            gds[j].wait()
            k, q = jobs[j]
            tag = j % _NBUF
            last_write[tag] = pltpu.async_copy(
                bufs[tag], out_hbm.at[pl.ds(row_off(k, q), _CHK)], wsems[tag])
        for tag in range(_NBUF):
            if last_write[tag] is not None:
                last_write[tag].wait()
                last_write[tag] = None

        for w, h in ((0, 0), (1, 1)):
            @pl.when(wid == w)
            def _(h=h):
                for p in range(_NBUF // 2):
                    q = h * (_NBUF // 2) + p
                    pltpu.async_copy(
                        x_hbm.at[idx_v.at[_TPW].at[0].at[pl.ds(_CHK * q, _CHK)]],
                        bufs[p], gsems[p]).wait()
                    pltpu.sync_copy(
                        bufs[p],
                        out_hbm.at[pl.ds((_ROWS - 1) * _B + _CHK * q, _CHK)])

    return gather_rows(x2, idx3)


def kernel(inputs):
    x = inputs
    x2 = jnp.transpose(x, (1, 0, 2)).reshape(_T * _B, _D)
    idx3 = jnp.asarray(_gather_indices())
    out2 = _sc_gather(x2, idx3)
    return jnp.transpose(out2.reshape(_ROWS, _B, _D), (1, 0, 2))


# speedup vs baseline: 6.1994x; 1.0525x over previous
import functools

import jax
import jax.numpy as jnp
import numpy as np
from jax import lax
from jax.experimental import pallas as pl
from jax.experimental.pallas import tpu as pltpu
from jax.experimental.pallas import tpu_sc as plsc

_PROB = 0.5
_NUM_PREFIX = 1
_B, _T, _D = 128, 577, 768
_NP = _T - _NUM_PREFIX
_NK = max(1, int(_NP * (1.0 - _PROB)))
_ROWS = _NUM_PREFIX + _NK
_NW = 32
_TPW = _ROWS // _NW
_NBUF = 2
_CHK = _B // _NBUF
_HB = _B // 2


@functools.lru_cache(maxsize=1)
def _gather_indices() -> np.ndarray:
    with jax.ensure_compile_time_eval():
        rand = jax.random.normal(jax.random.key(42), (_B, _NP), dtype=jnp.float32)
        order = jnp.argsort(rand, axis=-1)
        keep = jnp.sort(order[:, :_NK], axis=-1) + _NUM_PREFIX
        full = jnp.concatenate(
            [jnp.zeros((_B, _NUM_PREFIX), keep.dtype), keep], axis=1)
    rows = np.asarray(full).astype(np.int32)
    flat = rows.T * _B + np.arange(_B, dtype=np.int32)[None, :]
    return flat.reshape(_ROWS, 1, _B)


def _sc_gather(x2, idx3):
    mesh = plsc.VectorSubcoreMesh(core_axis_name="c", subcore_axis_name="s")

    @functools.partial(
        pl.kernel,
        mesh=mesh,
        out_type=jax.ShapeDtypeStruct((_ROWS * _B, _D), jnp.float32),
        scratch_types=[
            pltpu.VMEM((_TPW + 1, 1, _B), jnp.int32),
            *[pltpu.VMEM((_CHK, _D), jnp.float32) for _ in range(_NBUF)],
            *[pltpu.SemaphoreType.DMA for _ in range(2 * _NBUF)],
        ],
    )
    def gather_rows(x_hbm, idx_hbm, out_hbm, idx_v, *rest):
        bufs = rest[:_NBUF]
        gsems = rest[_NBUF:2 * _NBUF]
        wsems = rest[2 * _NBUF:]
        wid = lax.axis_index("s") * 2 + lax.axis_index("c")

        base_tok = wid * _TPW
        pltpu.sync_copy(idx_hbm.at[pl.ds(base_tok, _TPW)],
                        idx_v.at[pl.ds(0, _TPW)])
        @pl.when(wid >= _NW - 2)
        def _():
            pltpu.sync_copy(idx_hbm.at[_ROWS - 1], idx_v.at[_TPW])

        jobs = [(k, q) for k in range(_TPW) for q in range(_NBUF)]
        gds = [None] * len(jobs)
        last_write = [None] * _NBUF

        def row_off(k, q):
            return pl.multiple_of((base_tok + k) * _B + _CHK * q, _CHK)

        def start_gather(j):
            k, q = jobs[j]
            tag = j % _NBUF
            if last_write[tag] is not None:
                last_write[tag].wait()
                last_write[tag] = None
            gds[j] = pltpu.async_copy(
                x_hbm.at[idx_v.at[k].at[0].at[pl.ds(_CHK * q, _CHK)]],
                bufs[tag], gsems[tag])

        nj = len(jobs)
        for j in range(_NBUF - 1):
            start_gather(j)
        for j in range(nj):
            if j + _NBUF - 1 < nj:
                start_gather(j + _NBUF - 1)
            gds[j].wait()
            k, q = jobs[j]
            tag = j % _NBUF
            last_write[tag] = pltpu.async_copy(
                bufs[tag], out_hbm.at[pl.ds(row_off(k, q), _CHK)], wsems[tag])
        for tag in range(_NBUF):
            if last_write[tag] is not None:
                last_write[tag].wait()
                last_write[tag] = None

        for w, h in ((_NW - 2, 0), (_NW - 1, 1)):
            @pl.when(wid == w)
            def _(h=h):
                for p in range(_NBUF // 2):
                    q = h * (_NBUF // 2) + p
                    pltpu.async_copy(
                        x_hbm.at[idx_v.at[_TPW].at[0].at[pl.ds(_CHK * q, _CHK)]],
                        bufs[p], gsems[p]).wait()
                    pltpu.sync_copy(
                        bufs[p],
                        out_hbm.at[pl.ds((_ROWS - 1) * _B + _CHK * q, _CHK)])

    return gather_rows(x2, idx3)


def kernel(inputs):
    x = inputs
    x2 = jnp.transpose(x, (1, 0, 2)).reshape(_T * _B, _D)
    idx3 = jnp.asarray(_gather_indices())
    out2 = _sc_gather(x2, idx3)
    return jnp.transpose(out2.reshape(_ROWS, _B, _D), (1, 0, 2))
